# baseline (speedup 1.0000x reference)
import functools

import jax
import jax.numpy as jnp
from jax import lax
from jax.experimental import pallas as pl
from jax.experimental.pallas import tpu as pltpu
from jax.experimental.pallas import tpu_sc as plsc

N = 50000
N_PAD = 50176
E = 400000
E_PAD = 425984
HID = 32
L = 128
CPW = 104
K = 8
R = 4
D = 2
SUP = CPW // K
ROWS_PER_TILE = N_PAD // 16
ZROWS = 98
BT = 3136


def _tc_a_body(x_ref, wpl, wnl, wpr, wnr, bp, bn, hp, hn, basep, basen):
    xb = x_ref[...]
    hp[...] = jnp.dot(xb, wpl[...],
                      preferred_element_type=jnp.float32).astype(jnp.bfloat16)
    hn[...] = jnp.dot(xb, wnl[...],
                      preferred_element_type=jnp.float32).astype(jnp.bfloat16)
    basep[...] = jnp.dot(xb, wpr[...], preferred_element_type=jnp.float32) + bp[...]
    basen[...] = jnp.dot(xb, wnr[...], preferred_element_type=jnp.float32) + bn[...]


def _tc_b_body(sp, cp, sn, cn, basep, basen, zp, zn):
    cpv = jnp.maximum(cp[0, :, 0:1] + cp[1, :, 0:1], 1.0)
    cnv = jnp.maximum(cn[0, :, 0:1] + cn[1, :, 0:1], 1.0)
    spf = sp[0].astype(jnp.float32) + sp[1].astype(jnp.float32)
    snf = sn[0].astype(jnp.float32) + sn[1].astype(jnp.float32)
    zp[...] = jnp.tanh(spf / cpv + basep[...]).astype(jnp.bfloat16)
    zn[...] = jnp.tanh(snf / cnv + basen[...]).astype(jnp.bfloat16)


def _tc_c_body(p1, p2, p3, p4, cp, cn, zp, zn, w2pl, w2pr, b2p, w2nl, w2nr,
               b2n, out):
    cpv = jnp.maximum(cp[0, :, 0:1] + cp[1, :, 0:1], 1.0)
    cnv = jnp.maximum(cn[0, :, 0:1] + cn[1, :, 0:1], 1.0)
    op1 = (p1[0].astype(jnp.float32) + p1[1].astype(jnp.float32)) / cpv
    op2 = (p2[0].astype(jnp.float32) + p2[1].astype(jnp.float32)) / cnv
    on1 = (p3[0].astype(jnp.float32) + p3[1].astype(jnp.float32)) / cpv
    on2 = (p4[0].astype(jnp.float32) + p4[1].astype(jnp.float32)) / cnv
    zpb = zp[...].astype(jnp.float32)
    znb = zn[...].astype(jnp.float32)
    out_pos = (jnp.dot(jnp.concatenate([op1, op2], axis=1), w2pl[...],
                       preferred_element_type=jnp.float32)
               + jnp.dot(zpb, w2pr[...], preferred_element_type=jnp.float32)
               + b2p[...])
    out_neg = (jnp.dot(jnp.concatenate([on1, on2], axis=1), w2nl[...],
                       preferred_element_type=jnp.float32)
               + jnp.dot(znb, w2nr[...], preferred_element_type=jnp.float32)
               + b2n[...])
    out[...] = jnp.tanh(jnp.concatenate([out_pos, out_neg], axis=1))


def _row_spec(cols):
    return pl.BlockSpec((BT, cols), lambda i: (i, 0))


def _part_spec():
    return pl.BlockSpec((2, BT, HID), lambda i: (0, i, 0))


def _cnt_spec():
    return pl.BlockSpec((2, BT, 8), lambda i: (0, i, 0))


def _full_spec(r, c):
    return pl.BlockSpec((r, c), lambda i: (0, 0))


def _sc_helpers(cid, sid, acc, zeros_v, src_v, dst_v, msg_v, sem_g, sem_s):
    wid = cid * 16 + sid
    base = sid * ROWS_PER_TILE

    def zero_acc():
        for t in range(ROWS_PER_TILE // ZROWS):
            pltpu.sync_copy(zeros_v, acc.at[pl.ds(base + t * ZROWS, ZROWS)])

    def dump(out_ref):
        pltpu.sync_copy(acc.at[pl.ds(base, ROWS_PER_TILE)],
                        out_ref.at[cid, pl.ds(base, ROWS_PER_TILE)])

    def gather_pass(tab, srcm, dstm, out_ref):
        zero_acc()
        plsc.subcore_barrier()
        row0 = wid * CPW

        def fire(pp, j):
            b = j % R
            pltpu.async_copy(tab.at[src_v.at[pp, j]], msg_v.at[b],
                             sem_g.at[b])

        def retire(pp, j):
            b = j % R
            pltpu.make_async_copy(tab.at[src_v.at[pp, j]], msg_v.at[b],
                                  sem_g.at[b]).wait()
            pltpu.async_copy(msg_v.at[b], acc.at[dst_v.at[pp, j]],
                             sem_s.at[b], add=True)

        def wait_scat(pp, j):
            b = j % R
            pltpu.make_async_copy(msg_v.at[b], acc.at[dst_v.at[pp, j]],
                                  sem_s.at[b]).wait()

        def super_body(s, carry):
            p = lax.rem(s, 2)
            q = 1 - p
            r = row0 + s * K
            pltpu.sync_copy(srcm.at[pl.ds(r, K)], src_v.at[p])
            pltpu.sync_copy(dstm.at[pl.ds(r, K)], dst_v.at[p])
            for j in range(K):
                if j >= R:
                    wait_scat(p, j - R)
                else:
                    @pl.when(s > 0)
                    def _(jj=j):
                        wait_scat(q, jj - R + K)
                fire(p, j)
                if j >= D:
                    retire(p, j - D)
                else:
                    @pl.when(s > 0)
                    def _(jj=j):
                        retire(q, jj - D + K)
            return carry

        lax.fori_loop(0, SUP, super_body, 0)
        p_last = (SUP - 1) % 2
        for j in range(K - D, K):
            retire(p_last, j)
        for j in range(K - R, K):
            wait_scat(p_last, j)
        plsc.subcore_barrier()
        dump(out_ref)
        plsc.subcore_barrier()

    return wid, zero_acc, dump, gather_pass


def _sc_conv1_body(hp, hn, psrc, pdst, nsrc, ndst, zeros_h,
                   out_sp, out_sn,
                   acc, zeros_v, src_v, dst_v, msg_v, sem_g, sem_s):
    cid = lax.axis_index("c")
    sid = lax.axis_index("s")
    _, _, _, gather_pass = _sc_helpers(
        cid, sid, acc, zeros_v, src_v, dst_v, msg_v, sem_g, sem_s)

    pltpu.sync_copy(zeros_h, zeros_v)

    gather_pass(hp, psrc, pdst, out_sp)
    gather_pass(hn, nsrc, ndst, out_sn)


def _sc_conv2_body(zp, zn, psrc, pdst, nsrc, ndst, zeros_h,
                   out_p1, out_p2, out_p3, out_p4,
                   acc, zeros_v, src_v, dst_v, msg_v, sem_g, sem_s):
    cid = lax.axis_index("c")
    sid = lax.axis_index("s")
    _, _, _, gather_pass = _sc_helpers(
        cid, sid, acc, zeros_v, src_v, dst_v, msg_v, sem_g, sem_s)

    pltpu.sync_copy(zeros_h, zeros_v)

    gather_pass(zp, psrc, pdst, out_p1)
    gather_pass(zn, psrc, pdst, out_p3)
    gather_pass(zn, nsrc, ndst, out_p2)
    gather_pass(zp, nsrc, ndst, out_p4)


def _sc_counts_body(pdst, ndst, zeros8_h, ones8_h,
                    out_cp, out_cn,
                    acc8, zeros8_v, ones8_v, dst_v, sem):
    cid = lax.axis_index("c")
    sid = lax.axis_index("s")
    wid = cid * 16 + sid
    base = sid * ROWS_PER_TILE

    pltpu.sync_copy(zeros8_h, zeros8_v)
    pltpu.sync_copy(ones8_h, ones8_v)

    def count_pass(dstm, out_ref):
        for t in range(ROWS_PER_TILE // ZROWS):
            pltpu.sync_copy(zeros8_v, acc8.at[pl.ds(base + t * ZROWS, ZROWS)])
        plsc.subcore_barrier()
        row0 = wid * CPW

        def super_body(s, carry):
            @pl.when(s > 0)
            def _():
                for j in range(K):
                    pltpu.make_async_copy(ones8_v, acc8.at[dst_v.at[j]],
                                          sem).wait()
            r = row0 + s * K
            pltpu.sync_copy(dstm.at[pl.ds(r, K)], dst_v)
            for j in range(K):
                pltpu.async_copy(ones8_v, acc8.at[dst_v.at[j]], sem,
                                 add=True)
            return carry

        lax.fori_loop(0, SUP, super_body, 0)
        for j in range(K):
            pltpu.make_async_copy(ones8_v, acc8.at[dst_v.at[j]], sem).wait()
        plsc.subcore_barrier()
        pltpu.sync_copy(acc8.at[pl.ds(base, ROWS_PER_TILE)],
                        out_ref.at[cid, pl.ds(base, ROWS_PER_TILE)])
        plsc.subcore_barrier()

    count_pass(pdst, out_cp)
    count_pass(ndst, out_cn)


_SC_MESH = plsc.VectorSubcoreMesh(core_axis_name="c", subcore_axis_name="s")

_PART = jax.ShapeDtypeStruct((2, N_PAD, HID), jnp.bfloat16)
_CNT = jax.ShapeDtypeStruct((2, N_PAD, 8), jnp.float32)

_SC_PARAMS = pltpu.CompilerParams(use_tc_tiling_on_sc=False)

_GATHER_SCRATCH = [
    pltpu.VMEM_SHARED((N_PAD, HID), jnp.bfloat16),
    pltpu.VMEM((ZROWS, HID), jnp.bfloat16),
    pltpu.VMEM((2, K, L), jnp.int32),
    pltpu.VMEM((2, K, L), jnp.int32),
    pltpu.VMEM((R, L, HID), jnp.bfloat16),
    pltpu.SemaphoreType.DMA((R,)),
    pltpu.SemaphoreType.DMA((R,)),
]

_sc_conv1 = pl.kernel(
    _sc_conv1_body,
    out_type=[_PART, _PART],
    mesh=_SC_MESH,
    compiler_params=_SC_PARAMS,
    scratch_types=_GATHER_SCRATCH,
)

_sc_conv2 = pl.kernel(
    _sc_conv2_body,
    out_type=[_PART, _PART, _PART, _PART],
    mesh=_SC_MESH,
    compiler_params=_SC_PARAMS,
    scratch_types=_GATHER_SCRATCH,
)

_sc_counts = pl.kernel(
    _sc_counts_body,
    out_type=[_CNT, _CNT],
    mesh=_SC_MESH,
    compiler_params=_SC_PARAMS,
    scratch_types=[
        pltpu.VMEM_SHARED((N_PAD, 8), jnp.float32),
        pltpu.VMEM((ZROWS, 8), jnp.float32),
        pltpu.VMEM((L, 8), jnp.float32),
        pltpu.VMEM((K, L), jnp.int32),
        pltpu.SemaphoreType.DMA,
    ],
)


def _pad_edges(edge_index):
    pad_rows = N + (jnp.arange(E_PAD - E, dtype=jnp.int32) % (N_PAD - N))
    src = jnp.concatenate([edge_index[0], pad_rows])
    dst = jnp.concatenate([edge_index[1], pad_rows])
    return src.reshape(E_PAD // L, L), dst.reshape(E_PAD // L, L)


@jax.jit
def kernel(x, W1_pos_l, W1_pos_r, b1_pos_r, W1_neg_l, W1_neg_r, b1_neg_r,
           W2_pos_l, W2_pos_r, b2_pos_r, W2_neg_l, W2_neg_r, b2_neg_r,
           pos_edge_index, neg_edge_index):
    x_pad = jnp.pad(x, ((0, N_PAD - N), (0, 0)))
    psrc, pdst = _pad_edges(pos_edge_index)
    nsrc, ndst = _pad_edges(neg_edge_index)
    zeros_h = jnp.zeros((ZROWS, HID), jnp.bfloat16)
    zeros8_h = jnp.zeros((ZROWS, 8), jnp.float32)
    ones8_h = jnp.ones((L, 8), jnp.float32)

    cp, cn = _sc_counts(pdst, ndst, zeros8_h, ones8_h)

    grid = (N_PAD // BT,)
    hp, hn, basep, basen = pl.pallas_call(
        _tc_a_body,
        grid=grid,
        in_specs=[_row_spec(64)] + [_full_spec(64, HID)] * 4
                 + [_full_spec(1, HID)] * 2,
        out_specs=[_row_spec(HID)] * 4,
        out_shape=[jax.ShapeDtypeStruct((N_PAD, HID), jnp.bfloat16)] * 2
                  + [jax.ShapeDtypeStruct((N_PAD, HID), jnp.float32)] * 2,
    )(x_pad, W1_pos_l, W1_neg_l, W1_pos_r, W1_neg_r,
      b1_pos_r.reshape(1, HID), b1_neg_r.reshape(1, HID))

    sp, sn = _sc_conv1(hp, hn, psrc, pdst, nsrc, ndst, zeros_h)
    sp = sp.astype(jnp.float32)
    sn = sn.astype(jnp.float32)

    zp, zn = pl.pallas_call(
        _tc_b_body,
        grid=grid,
        in_specs=[_part_spec(), _cnt_spec(), _part_spec(), _cnt_spec(),
                  _row_spec(HID), _row_spec(HID)],
        out_specs=[_row_spec(HID)] * 2,
        out_shape=[jax.ShapeDtypeStruct((N_PAD, HID), jnp.bfloat16)] * 2,
    )(sp, cp, sn, cn, basep, basen)

    p1, p2, p3, p4 = _sc_conv2(zp, zn, psrc, pdst, nsrc, ndst, zeros_h)
    p1 = p1.astype(jnp.float32)
    p2 = p2.astype(jnp.float32)
    p3 = p3.astype(jnp.float32)
    p4 = p4.astype(jnp.float32)

    out = pl.pallas_call(
        _tc_c_body,
        grid=grid,
        in_specs=[_part_spec()] * 4 + [_cnt_spec()] * 2 + [_row_spec(HID)] * 2
                 + [_full_spec(2 * HID, HID), _full_spec(HID, HID),
                    _full_spec(1, HID)] * 2,
        out_specs=_row_spec(2 * HID),
        out_shape=jax.ShapeDtypeStruct((N_PAD, 2 * HID), jnp.float32),
    )(p1, p2, p3, p4, cp, cn, zp, zn,
      W2_pos_l, W2_pos_r, b2_pos_r.reshape(1, HID),
      W2_neg_l, W2_neg_r, b2_neg_r.reshape(1, HID))

    return out[:N]


# speedup vs baseline: 1.1880x; 1.1880x over previous
import functools

import jax
import jax.numpy as jnp
from jax import lax
from jax.experimental import pallas as pl
from jax.experimental.pallas import tpu as pltpu
from jax.experimental.pallas import tpu_sc as plsc

N = 50000
N_PAD = 50176
E = 400000
E_PAD = 425984
HID = 32
L = 128
CPW = 104
K = 8
R = 4
D = 2
SUP = CPW // K
ROWS_PER_TILE = N_PAD // 16
ZROWS = 98
BT = 3136


def _tc_a_body(x_ref, wpl, wnl, wpr, wnr, bp, bn, hp, hn, basep, basen):
    xb = x_ref[...]
    hp[...] = jnp.dot(xb, wpl[...],
                      preferred_element_type=jnp.float32).astype(jnp.bfloat16)
    hn[...] = jnp.dot(xb, wnl[...],
                      preferred_element_type=jnp.float32).astype(jnp.bfloat16)
    basep[...] = jnp.dot(xb, wpr[...], preferred_element_type=jnp.float32) + bp[...]
    basen[...] = jnp.dot(xb, wnr[...], preferred_element_type=jnp.float32) + bn[...]


def _tc_b_body(sp, cp, sn, cn, basep, basen, zp, zn):
    cpv = jnp.maximum(cp[0, :, 0:1] + cp[1, :, 0:1], 1.0)
    cnv = jnp.maximum(cn[0, :, 0:1] + cn[1, :, 0:1], 1.0)
    spf = sp[0].astype(jnp.float32) + sp[1].astype(jnp.float32)
    snf = sn[0].astype(jnp.float32) + sn[1].astype(jnp.float32)
    zp[...] = jnp.tanh(spf / cpv + basep[...]).astype(jnp.bfloat16)
    zn[...] = jnp.tanh(snf / cnv + basen[...]).astype(jnp.bfloat16)


def _tc_c_body(p1, p2, p3, p4, cp, cn, zp, zn, w2pl, w2pr, b2p, w2nl, w2nr,
               b2n, out):
    cpv = jnp.maximum(cp[0, :, 0:1] + cp[1, :, 0:1], 1.0)
    cnv = jnp.maximum(cn[0, :, 0:1] + cn[1, :, 0:1], 1.0)
    op1 = (p1[0].astype(jnp.float32) + p1[1].astype(jnp.float32)) / cpv
    op2 = (p2[0].astype(jnp.float32) + p2[1].astype(jnp.float32)) / cnv
    on1 = (p3[0].astype(jnp.float32) + p3[1].astype(jnp.float32)) / cpv
    on2 = (p4[0].astype(jnp.float32) + p4[1].astype(jnp.float32)) / cnv
    zpb = zp[...].astype(jnp.float32)
    znb = zn[...].astype(jnp.float32)
    out_pos = (jnp.dot(jnp.concatenate([op1, op2], axis=1), w2pl[...],
                       preferred_element_type=jnp.float32)
               + jnp.dot(zpb, w2pr[...], preferred_element_type=jnp.float32)
               + b2p[...])
    out_neg = (jnp.dot(jnp.concatenate([on1, on2], axis=1), w2nl[...],
                       preferred_element_type=jnp.float32)
               + jnp.dot(znb, w2nr[...], preferred_element_type=jnp.float32)
               + b2n[...])
    out[...] = jnp.tanh(jnp.concatenate([out_pos, out_neg], axis=1))


def _row_spec(cols):
    return pl.BlockSpec((BT, cols), lambda i: (i, 0))


def _part_spec():
    return pl.BlockSpec((2, BT, HID), lambda i: (0, i, 0))


def _cnt_spec():
    return pl.BlockSpec((2, BT, 8), lambda i: (0, i, 0))


def _full_spec(r, c):
    return pl.BlockSpec((r, c), lambda i: (0, 0))


def _sc_helpers(cid, sid, acc, zeros_v, src_v, dst_v, msg_v, sem_g, sem_s):
    wid = cid * 16 + sid
    base = sid * ROWS_PER_TILE

    def zero_acc():
        for t in range(ROWS_PER_TILE // ZROWS):
            pltpu.sync_copy(zeros_v, acc.at[pl.ds(base + t * ZROWS, ZROWS)])

    def dump(out_ref):
        pltpu.sync_copy(acc.at[pl.ds(base, ROWS_PER_TILE)],
                        out_ref.at[cid, pl.ds(base, ROWS_PER_TILE)])

    def gather_pass(tab, srcm, dstm, out_ref):
        zero_acc()
        plsc.subcore_barrier()
        row0 = wid * CPW

        def fire(pp, j):
            b = j % R
            pltpu.async_copy(tab.at[src_v.at[pp, j]], msg_v.at[b],
                             sem_g.at[b])

        def retire(pp, j):
            b = j % R
            pltpu.make_async_copy(tab.at[src_v.at[pp, j]], msg_v.at[b],
                                  sem_g.at[b]).wait()
            pltpu.async_copy(msg_v.at[b], acc.at[dst_v.at[pp, j]],
                             sem_s.at[b], add=True)

        def wait_scat(pp, j):
            b = j % R
            pltpu.make_async_copy(msg_v.at[b], acc.at[dst_v.at[pp, j]],
                                  sem_s.at[b]).wait()

        def super_body(s, carry):
            p = lax.rem(s, 2)
            q = 1 - p
            r = row0 + s * K
            pltpu.sync_copy(srcm.at[pl.ds(r, K)], src_v.at[p])
            pltpu.sync_copy(dstm.at[pl.ds(r, K)], dst_v.at[p])
            for j in range(K):
                if j >= R:
                    wait_scat(p, j - R)
                else:
                    @pl.when(s > 0)
                    def _(jj=j):
                        wait_scat(q, jj - R + K)
                fire(p, j)
                if j >= D:
                    retire(p, j - D)
                else:
                    @pl.when(s > 0)
                    def _(jj=j):
                        retire(q, jj - D + K)
            return carry

        lax.fori_loop(0, SUP, super_body, 0)
        p_last = (SUP - 1) % 2
        for j in range(K - D, K):
            retire(p_last, j)
        for j in range(K - R, K):
            wait_scat(p_last, j)
        plsc.subcore_barrier()
        dump(out_ref)
        plsc.subcore_barrier()

    return wid, zero_acc, dump, gather_pass


def _sc_conv1_body(hp, hn, psrc, pdst, nsrc, ndst, zeros_h,
                   out_sp, out_sn,
                   acc, zeros_v, src_v, dst_v, msg_v, sem_g, sem_s):
    cid = lax.axis_index("c")
    sid = lax.axis_index("s")
    _, _, _, gather_pass = _sc_helpers(
        cid, sid, acc, zeros_v, src_v, dst_v, msg_v, sem_g, sem_s)

    pltpu.sync_copy(zeros_h, zeros_v)

    gather_pass(hp, psrc, pdst, out_sp)
    gather_pass(hn, nsrc, ndst, out_sn)


def _sc_conv2_body(zp, zn, psrc, pdst, nsrc, ndst, zeros_h,
                   out_p1, out_p2, out_p3, out_p4,
                   acc, zeros_v, src_v, dst_v, msg_v, sem_g, sem_s):
    cid = lax.axis_index("c")
    sid = lax.axis_index("s")
    _, _, _, gather_pass = _sc_helpers(
        cid, sid, acc, zeros_v, src_v, dst_v, msg_v, sem_g, sem_s)

    pltpu.sync_copy(zeros_h, zeros_v)

    gather_pass(zp, psrc, pdst, out_p1)
    gather_pass(zn, psrc, pdst, out_p3)
    gather_pass(zn, nsrc, ndst, out_p2)
    gather_pass(zp, nsrc, ndst, out_p4)


def _sc_counts_body(pdst, ndst, zeros8_h, ones8_h,
                    out_cp, out_cn,
                    acc8, zeros8_v, ones8_v, dst_v, sem):
    cid = lax.axis_index("c")
    sid = lax.axis_index("s")
    wid = cid * 16 + sid
    base = sid * ROWS_PER_TILE

    pltpu.sync_copy(zeros8_h, zeros8_v)
    pltpu.sync_copy(ones8_h, ones8_v)

    def count_pass(dstm, out_ref):
        for t in range(ROWS_PER_TILE // ZROWS):
            pltpu.sync_copy(zeros8_v, acc8.at[pl.ds(base + t * ZROWS, ZROWS)])
        plsc.subcore_barrier()
        row0 = wid * CPW

        def super_body(s, carry):
            @pl.when(s > 0)
            def _():
                for j in range(K):
                    pltpu.make_async_copy(ones8_v, acc8.at[dst_v.at[j]],
                                          sem).wait()
            r = row0 + s * K
            pltpu.sync_copy(dstm.at[pl.ds(r, K)], dst_v)
            for j in range(K):
                pltpu.async_copy(ones8_v, acc8.at[dst_v.at[j]], sem,
                                 add=True)
            return carry

        lax.fori_loop(0, SUP, super_body, 0)
        for j in range(K):
            pltpu.make_async_copy(ones8_v, acc8.at[dst_v.at[j]], sem).wait()
        plsc.subcore_barrier()
        pltpu.sync_copy(acc8.at[pl.ds(base, ROWS_PER_TILE)],
                        out_ref.at[cid, pl.ds(base, ROWS_PER_TILE)])
        plsc.subcore_barrier()

    count_pass(pdst, out_cp)
    count_pass(ndst, out_cn)


_SC_MESH = plsc.VectorSubcoreMesh(core_axis_name="c", subcore_axis_name="s")

_PART = jax.ShapeDtypeStruct((2, N_PAD, HID), jnp.bfloat16)
_CNT = jax.ShapeDtypeStruct((2, N_PAD, 8), jnp.float32)

_SC_PARAMS = pltpu.CompilerParams(use_tc_tiling_on_sc=False)

_GATHER_SCRATCH = [
    pltpu.VMEM_SHARED((N_PAD, HID), jnp.bfloat16),
    pltpu.VMEM((ZROWS, HID), jnp.bfloat16),
    pltpu.VMEM((2, K, L), jnp.int32),
    pltpu.VMEM((2, K, L), jnp.int32),
    pltpu.VMEM((R, L, HID), jnp.bfloat16),
    pltpu.SemaphoreType.DMA((R,)),
    pltpu.SemaphoreType.DMA((R,)),
]

_sc_conv1 = pl.kernel(
    _sc_conv1_body,
    out_type=[_PART, _PART],
    mesh=_SC_MESH,
    compiler_params=_SC_PARAMS,
    scratch_types=_GATHER_SCRATCH,
)

_sc_conv2 = pl.kernel(
    _sc_conv2_body,
    out_type=[_PART, _PART, _PART, _PART],
    mesh=_SC_MESH,
    compiler_params=_SC_PARAMS,
    scratch_types=_GATHER_SCRATCH,
)

_sc_counts = pl.kernel(
    _sc_counts_body,
    out_type=[_CNT, _CNT],
    mesh=_SC_MESH,
    compiler_params=_SC_PARAMS,
    scratch_types=[
        pltpu.VMEM_SHARED((N_PAD, 8), jnp.float32),
        pltpu.VMEM((ZROWS, 8), jnp.float32),
        pltpu.VMEM((L, 8), jnp.float32),
        pltpu.VMEM((K, L), jnp.int32),
        pltpu.SemaphoreType.DMA,
    ],
)


def _pad_edges(edge_index):
    pad_rows = N + (jnp.arange(E_PAD - E, dtype=jnp.int32) % (N_PAD - N))
    src = jnp.concatenate([edge_index[0], pad_rows])
    dst = jnp.concatenate([edge_index[1], pad_rows])
    return src.reshape(E_PAD // L, L), dst.reshape(E_PAD // L, L)


@jax.jit
def kernel(x, W1_pos_l, W1_pos_r, b1_pos_r, W1_neg_l, W1_neg_r, b1_neg_r,
           W2_pos_l, W2_pos_r, b2_pos_r, W2_neg_l, W2_neg_r, b2_neg_r,
           pos_edge_index, neg_edge_index):
    x_pad = jnp.pad(x, ((0, N_PAD - N), (0, 0)))
    psrc, pdst = _pad_edges(pos_edge_index)
    nsrc, ndst = _pad_edges(neg_edge_index)
    zeros_h = jnp.zeros((ZROWS, HID), jnp.bfloat16)
    zeros8_h = jnp.zeros((ZROWS, 8), jnp.float32)
    ones8_h = jnp.ones((L, 8), jnp.float32)

    cp, cn = _sc_counts(pdst, ndst, zeros8_h, ones8_h)

    grid = (N_PAD // BT,)
    hp, hn, basep, basen = pl.pallas_call(
        _tc_a_body,
        grid=grid,
        in_specs=[_row_spec(64)] + [_full_spec(64, HID)] * 4
                 + [_full_spec(1, HID)] * 2,
        out_specs=[_row_spec(HID)] * 4,
        out_shape=[jax.ShapeDtypeStruct((N_PAD, HID), jnp.bfloat16)] * 2
                  + [jax.ShapeDtypeStruct((N_PAD, HID), jnp.float32)] * 2,
    )(x_pad, W1_pos_l, W1_neg_l, W1_pos_r, W1_neg_r,
      b1_pos_r.reshape(1, HID), b1_neg_r.reshape(1, HID))

    sp, sn = _sc_conv1(hp, hn, psrc, pdst, nsrc, ndst, zeros_h)

    zp, zn = pl.pallas_call(
        _tc_b_body,
        grid=grid,
        in_specs=[_part_spec(), _cnt_spec(), _part_spec(), _cnt_spec(),
                  _row_spec(HID), _row_spec(HID)],
        out_specs=[_row_spec(HID)] * 2,
        out_shape=[jax.ShapeDtypeStruct((N_PAD, HID), jnp.bfloat16)] * 2,
    )(sp, cp, sn, cn, basep, basen)

    p1, p2, p3, p4 = _sc_conv2(zp, zn, psrc, pdst, nsrc, ndst, zeros_h)

    out = pl.pallas_call(
        _tc_c_body,
        grid=grid,
        in_specs=[_part_spec()] * 4 + [_cnt_spec()] * 2 + [_row_spec(HID)] * 2
                 + [_full_spec(2 * HID, HID), _full_spec(HID, HID),
                    _full_spec(1, HID)] * 2,
        out_specs=_row_spec(2 * HID),
        out_shape=jax.ShapeDtypeStruct((N_PAD, 2 * HID), jnp.float32),
    )(p1, p2, p3, p4, cp, cn, zp, zn,
      W2_pos_l, W2_pos_r, b2_pos_r.reshape(1, HID),
      W2_neg_l, W2_neg_r, b2_neg_r.reshape(1, HID))

    return out[:N]
